# baseline (speedup 1.0000x reference)
import functools

import jax
import jax.numpy as jnp
from jax import lax
from jax.experimental import pallas as pl
from jax.experimental.pallas import tpu as pltpu
from jax.experimental.pallas import tpu_sc as plsc

N = 10000
NP = 10240
E = 320000
D1 = 128
D2 = 64
BLK = 512

NC, NS, L = 2, 16, 16
NW = NC * NS
CH = 128
CPT = 80
EPAD = NW * CPT * CH
DUMP = NP - 1
RPT = NP // NS

_mesh = lambda: plsc.VectorSubcoreMesh(core_axis_name="c", subcore_axis_name="s")


def _deg_body(dst_hbm, out_hbm, dst_v, ones_v, acc_sp, *_):
    cid = lax.axis_index("c")
    sid = lax.axis_index("s")
    wid = cid * NS + sid

    ones = jnp.full((L,), 1.0, dtype=jnp.float32)

    def _fill(r, _):
        ones_v[r, pl.ds(0, L)] = ones
        return _
    lax.fori_loop(0, CH, _fill, None)

    for k in range(RPT // CH):
        pltpu.sync_copy(ones_v, acc_sp.at[pl.ds(sid * RPT + k * CH, CH)])

    pltpu.sync_copy(dst_hbm.at[wid], dst_v)
    plsc.subcore_barrier()

    def _chunk(c, _):
        pltpu.sync_copy(ones_v, acc_sp.at[dst_v.at[c]], add=True)
        return _
    lax.fori_loop(0, CPT, _chunk, None)
    plsc.subcore_barrier()

    pltpu.sync_copy(acc_sp.at[pl.ds(sid * RPT, RPT)],
                    out_hbm.at[cid, pl.ds(sid * RPT, RPT)])


def _deg_sc(dst3):
    k = pl.kernel(
        _deg_body,
        out_type=jax.ShapeDtypeStruct((NC, NP, L), jnp.float32),
        mesh=_mesh(),
        scratch_types=[
            pltpu.VMEM((CPT, CH), jnp.int32),
            pltpu.VMEM((CH, L), jnp.float32),
            pltpu.VMEM_SHARED((NP, L), jnp.float32),
        ],
    )
    return k(dst3)


def _mm1_body(x_ref, w_ref, b_ref, d0_ref, d1_ref, o_ref, dv_ref):
    i = pl.program_id(0)
    deg = d0_ref[:, 0:1] + d1_ref[:, 0:1]
    rows = i * BLK + lax.broadcasted_iota(jnp.int32, (BLK, 1), 0)
    dcol = jnp.where(rows < N, lax.rsqrt(deg - 1.0), 0.0)
    dv = jnp.broadcast_to(dcol, (BLK, D1))
    dv_ref[...] = dv
    acc = jnp.dot(x_ref[...], w_ref[...], preferred_element_type=jnp.float32)
    o_ref[...] = (acc + b_ref[0:1, :]) * dv


def _mm1_tc(xp, W1, b1, deg0, deg1):
    b1b = jnp.broadcast_to(b1[None, :], (8, D1))
    return pl.pallas_call(
        _mm1_body,
        grid=(NP // BLK,),
        in_specs=[
            pl.BlockSpec((BLK, D1), lambda i: (i, 0)),
            pl.BlockSpec((D1, D1), lambda i: (0, 0)),
            pl.BlockSpec((8, D1), lambda i: (0, 0)),
            pl.BlockSpec((BLK, L), lambda i: (i, 0)),
            pl.BlockSpec((BLK, L), lambda i: (i, 0)),
        ],
        out_specs=[
            pl.BlockSpec((BLK, D1), lambda i: (i, 0)),
            pl.BlockSpec((BLK, D1), lambda i: (i, 0)),
        ],
        out_shape=[
            jax.ShapeDtypeStruct((NP, D1), jnp.float32),
            jax.ShapeDtypeStruct((NP, D1), jnp.float32),
        ],
    )(xp, W1, b1b, deg0, deg1)


CPT0 = 96
CPT1 = 64
G = 32
TOTC = EPAD // CH


def _spmm_body(D, table_hbm, src_hbm, dst_hbm, out_hbm, src_v, dst_v,
               rows0_v, rows1_v, acc_sp, sem0, sem1, *_):
    cid = lax.axis_index("c")
    sid = lax.axis_index("s")

    zeros = jnp.zeros((L,), dtype=jnp.float32)

    def _zfill(r, _):
        for c in range(D // L):
            rows0_v[r, pl.ds(c * L, L)] = zeros
        return _
    lax.fori_loop(0, CH, _zfill, None)
    for k in range(RPT // CH):
        pltpu.sync_copy(rows0_v, acc_sp.at[pl.ds(sid * RPT + k * CH, CH)])
    plsc.subcore_barrier()

    n_groups = jnp.where(cid == 0, CPT0 // G, CPT1 // G)
    base = jnp.where(cid == 0, sid * CPT0, NS * CPT0 + sid * CPT1)

    def _group(g, _):
        pltpu.sync_copy(src_hbm.at[pl.ds(base + g * G, G)], src_v)
        pltpu.sync_copy(dst_hbm.at[pl.ds(base + g * G, G)], dst_v)
        pltpu.async_copy(table_hbm.at[src_v.at[0]], rows0_v, sem0)

        def _pair(i, _):
            a = 2 * i
            pltpu.make_async_copy(table_hbm.at[src_v.at[a]],
                                  rows0_v, sem0).wait()
            pltpu.async_copy(table_hbm.at[src_v.at[a + 1]], rows1_v, sem1)
            pltpu.sync_copy(rows0_v, acc_sp.at[dst_v.at[a]], add=True)
            pltpu.make_async_copy(table_hbm.at[src_v.at[a + 1]],
                                  rows1_v, sem1).wait()

            @pl.when(i < G // 2 - 1)
            def _():
                pltpu.async_copy(table_hbm.at[src_v.at[a + 2]], rows0_v, sem0)

            pltpu.sync_copy(rows1_v, acc_sp.at[dst_v.at[a + 1]], add=True)
            return _
        lax.fori_loop(0, G // 2, _pair, None)
        return _
    lax.fori_loop(0, n_groups, _group, None)
    plsc.subcore_barrier()

    pltpu.sync_copy(acc_sp.at[pl.ds(sid * RPT, RPT)],
                    out_hbm.at[cid, pl.ds(sid * RPT, RPT)])


def _spmm_sc(table, srcC, dstC, D):
    k = pl.kernel(
        functools.partial(_spmm_body, D),
        out_type=jax.ShapeDtypeStruct((NC, NP, D), jnp.float32),
        mesh=_mesh(),
        compiler_params=pltpu.CompilerParams(use_tc_tiling_on_sc=False)
        if D != 128 else None,
        scratch_types=[
            pltpu.VMEM((G, CH), jnp.int32),
            pltpu.VMEM((G, CH), jnp.int32),
            pltpu.VMEM((CH, D), jnp.float32),
            pltpu.VMEM((CH, D), jnp.float32),
            pltpu.VMEM_SHARED((NP, D), jnp.float32),
            pltpu.SemaphoreType.DMA,
            pltpu.SemaphoreType.DMA,
        ],
    )
    return k(table, srcC, dstC)


def _mm2_body(a_ref, b_ref, t_ref, w_ref, bias_ref, dv_ref, o_ref):
    dv = dv_ref[...]
    h = jnp.maximum((a_ref[...] + b_ref[...] + t_ref[...]) * dv, 0.0)
    acc = jnp.dot(h, w_ref[...], preferred_element_type=jnp.float32)
    o_ref[...] = (acc + bias_ref[0:1, :]) * dv[:, :D2]


def _mm2_tc(acc0, acc1, table1, W2, b2, dinv_b):
    b2b = jnp.broadcast_to(b2[None, :], (8, D2))
    return pl.pallas_call(
        _mm2_body,
        grid=(NP // BLK,),
        in_specs=[
            pl.BlockSpec((BLK, D1), lambda i: (i, 0)),
            pl.BlockSpec((BLK, D1), lambda i: (i, 0)),
            pl.BlockSpec((BLK, D1), lambda i: (i, 0)),
            pl.BlockSpec((D1, D2), lambda i: (0, 0)),
            pl.BlockSpec((8, D2), lambda i: (0, 0)),
            pl.BlockSpec((BLK, D1), lambda i: (i, 0)),
        ],
        out_specs=pl.BlockSpec((BLK, D2), lambda i: (i, 0)),
        out_shape=jax.ShapeDtypeStruct((NP, D2), jnp.float32),
    )(acc0, acc1, table1, W2, b2b, dinv_b)


FBLK = 200


def _fin_body(a_ref, b_ref, t_ref, dv_ref, o_ref):
    o_ref[...] = (a_ref[...] + b_ref[...] + t_ref[...]) * dv_ref[:, :D2]


def _fin_tc(acc0, acc1, table2, dinv_b):
    return pl.pallas_call(
        _fin_body,
        grid=(N // FBLK,),
        in_specs=[
            pl.BlockSpec((FBLK, D2), lambda i: (i, 0)),
            pl.BlockSpec((FBLK, D2), lambda i: (i, 0)),
            pl.BlockSpec((FBLK, D2), lambda i: (i, 0)),
            pl.BlockSpec((FBLK, D1), lambda i: (i, 0)),
        ],
        out_specs=pl.BlockSpec((FBLK, D2), lambda i: (i, 0)),
        out_shape=jax.ShapeDtypeStruct((N, D2), jnp.float32),
    )(acc0, acc1, table2, dinv_b)


@jax.jit
def kernel(x, edge_index, W1, b1, W2, b2):
    src = edge_index[0].astype(jnp.int32)
    dst = edge_index[1].astype(jnp.int32)

    pad = EPAD - E
    src_p = jnp.concatenate([src, jnp.full((pad,), DUMP, jnp.int32)])
    dst_p = jnp.concatenate([dst, jnp.full((pad,), DUMP, jnp.int32)])
    dst3 = dst_p.reshape(NW, CPT, CH)
    srcC = src_p.reshape(TOTC, CH)
    dstC = dst_p.reshape(TOTC, CH)

    cnt = _deg_sc(dst3)
    table1, dinv_b = _mm1_tc(x, W1, b1, cnt[0], cnt[1])
    acc1 = _spmm_sc(table1, srcC, dstC, D1)
    table2 = _mm2_tc(acc1[0], acc1[1], table1, W2, b2, dinv_b)
    acc2 = _spmm_sc(table2, srcC, dstC, D2)
    return _fin_tc(acc2[0], acc2[1], table2, dinv_b)


# speedup vs baseline: 1.0713x; 1.0713x over previous
import functools

import jax
import jax.numpy as jnp
from jax import lax
from jax.experimental import pallas as pl
from jax.experimental.pallas import tpu as pltpu
from jax.experimental.pallas import tpu_sc as plsc

N = 10000
NP = 10240
E = 320000
D1 = 128
D2 = 64
BLK = 256

NC, NS, L = 2, 16, 16
NW = NC * NS
CH = 128
CPT = 80
EPAD = NW * CPT * CH
DUMP = NP - 1
RPT = NP // NS

_mesh = lambda: plsc.VectorSubcoreMesh(core_axis_name="c", subcore_axis_name="s")


def _deg_body(dst_hbm, out_hbm, dst_v, ones_v, acc_sp, *_):
    cid = lax.axis_index("c")
    sid = lax.axis_index("s")
    wid = cid * NS + sid

    ones = jnp.full((L,), 1.0, dtype=jnp.float32)

    def _fill(r, _):
        ones_v[r, pl.ds(0, L)] = ones
        return _
    lax.fori_loop(0, CH, _fill, None)

    for k in range(RPT // CH):
        pltpu.sync_copy(ones_v, acc_sp.at[pl.ds(sid * RPT + k * CH, CH)])

    pltpu.sync_copy(dst_hbm.at[wid], dst_v)
    plsc.subcore_barrier()

    def _chunk(c, _):
        pltpu.sync_copy(ones_v, acc_sp.at[dst_v.at[c]], add=True)
        return _
    lax.fori_loop(0, CPT, _chunk, None)
    plsc.subcore_barrier()

    pltpu.sync_copy(acc_sp.at[pl.ds(sid * RPT, RPT)],
                    out_hbm.at[cid, pl.ds(sid * RPT, RPT)])


def _deg_sc(dst3):
    k = pl.kernel(
        _deg_body,
        out_type=jax.ShapeDtypeStruct((NC, NP, L), jnp.float32),
        mesh=_mesh(),
        scratch_types=[
            pltpu.VMEM((CPT, CH), jnp.int32),
            pltpu.VMEM((CH, L), jnp.float32),
            pltpu.VMEM_SHARED((NP, L), jnp.float32),
        ],
    )
    return k(dst3)


def _mm1_body(x_ref, w_ref, b_ref, d0_ref, d1_ref, o_ref, dv_ref):
    i = pl.program_id(0)
    deg = d0_ref[:, 0:1] + d1_ref[:, 0:1]
    rows = i * BLK + lax.broadcasted_iota(jnp.int32, (BLK, 1), 0)
    dcol = jnp.where(rows < N, lax.rsqrt(deg - 1.0), 0.0)
    dv = jnp.broadcast_to(dcol, (BLK, D1))
    dv_ref[...] = dv
    acc = jnp.dot(x_ref[...], w_ref[...], preferred_element_type=jnp.float32)
    o_ref[...] = (acc + b_ref[0:1, :]) * dv


def _mm1_tc(xp, W1, b1, deg0, deg1):
    b1b = jnp.broadcast_to(b1[None, :], (8, D1))
    return pl.pallas_call(
        _mm1_body,
        grid=(NP // BLK,),
        in_specs=[
            pl.BlockSpec((BLK, D1), lambda i: (i, 0)),
            pl.BlockSpec((D1, D1), lambda i: (0, 0)),
            pl.BlockSpec((8, D1), lambda i: (0, 0)),
            pl.BlockSpec((BLK, L), lambda i: (i, 0)),
            pl.BlockSpec((BLK, L), lambda i: (i, 0)),
        ],
        out_specs=[
            pl.BlockSpec((BLK, D1), lambda i: (i, 0)),
            pl.BlockSpec((BLK, D1), lambda i: (i, 0)),
        ],
        out_shape=[
            jax.ShapeDtypeStruct((NP, D1), jnp.float32),
            jax.ShapeDtypeStruct((NP, D1), jnp.float32),
        ],
    )(xp, W1, b1b, deg0, deg1)


CPT0 = 96
CPT1 = 64
G = 32
TOTC = EPAD // CH


def _spmm_body(D, table_hbm, src_hbm, dst_hbm, out_hbm, src_v, dst_v,
               rows0_v, rows1_v, acc_sp, sem0, sem1, *_):
    cid = lax.axis_index("c")
    sid = lax.axis_index("s")

    zeros = jnp.zeros((L,), dtype=jnp.float32)

    def _zfill(r, _):
        for c in range(D // L):
            rows0_v[r, pl.ds(c * L, L)] = zeros
        return _
    lax.fori_loop(0, CH, _zfill, None)
    for k in range(RPT // CH):
        pltpu.sync_copy(rows0_v, acc_sp.at[pl.ds(sid * RPT + k * CH, CH)])
    plsc.subcore_barrier()

    n_groups = jnp.where(cid == 0, CPT0 // G, CPT1 // G)
    base = jnp.where(cid == 0, sid * CPT0, NS * CPT0 + sid * CPT1)

    def _group(g, _):
        pltpu.sync_copy(src_hbm.at[pl.ds(base + g * G, G)], src_v)
        pltpu.sync_copy(dst_hbm.at[pl.ds(base + g * G, G)], dst_v)
        pltpu.async_copy(table_hbm.at[src_v.at[0]], rows0_v, sem0)

        def _pair(i, _):
            a = 2 * i
            pltpu.make_async_copy(table_hbm.at[src_v.at[a]],
                                  rows0_v, sem0).wait()
            pltpu.async_copy(table_hbm.at[src_v.at[a + 1]], rows1_v, sem1)
            pltpu.sync_copy(rows0_v, acc_sp.at[dst_v.at[a]], add=True)
            pltpu.make_async_copy(table_hbm.at[src_v.at[a + 1]],
                                  rows1_v, sem1).wait()

            @pl.when(i < G // 2 - 1)
            def _():
                pltpu.async_copy(table_hbm.at[src_v.at[a + 2]], rows0_v, sem0)

            pltpu.sync_copy(rows1_v, acc_sp.at[dst_v.at[a + 1]], add=True)
            return _
        lax.fori_loop(0, G // 2, _pair, None)
        return _
    lax.fori_loop(0, n_groups, _group, None)
    plsc.subcore_barrier()

    pltpu.sync_copy(acc_sp.at[pl.ds(sid * RPT, RPT)],
                    out_hbm.at[cid, pl.ds(sid * RPT, RPT)])


def _spmm_sc(table, srcC, dstC, D):
    k = pl.kernel(
        functools.partial(_spmm_body, D),
        out_type=jax.ShapeDtypeStruct((NC, NP, D), jnp.float32),
        mesh=_mesh(),
        compiler_params=pltpu.CompilerParams(use_tc_tiling_on_sc=False),
        scratch_types=[
            pltpu.VMEM((G, CH), jnp.int32),
            pltpu.VMEM((G, CH), jnp.int32),
            pltpu.VMEM((CH, D), jnp.float32),
            pltpu.VMEM((CH, D), jnp.float32),
            pltpu.VMEM_SHARED((NP, D), jnp.float32),
            pltpu.SemaphoreType.DMA,
            pltpu.SemaphoreType.DMA,
        ],
    )
    return k(table, srcC, dstC)


def _mm2_body(a_ref, b_ref, t_ref, w_ref, bias_ref, dv_ref, o_ref):
    dv = dv_ref[...]
    h = jnp.maximum((a_ref[...] + b_ref[...] + t_ref[...]) * dv, 0.0)
    acc = jnp.dot(h, w_ref[...], preferred_element_type=jnp.float32)
    o_ref[...] = (acc + bias_ref[0:1, :]) * dv[:, :D2]


def _mm2_tc(acc0, acc1, table1, W2, b2, dinv_b):
    b2b = jnp.broadcast_to(b2[None, :], (8, D2))
    return pl.pallas_call(
        _mm2_body,
        grid=(NP // BLK,),
        in_specs=[
            pl.BlockSpec((BLK, D1), lambda i: (i, 0)),
            pl.BlockSpec((BLK, D1), lambda i: (i, 0)),
            pl.BlockSpec((BLK, D1), lambda i: (i, 0)),
            pl.BlockSpec((D1, D2), lambda i: (0, 0)),
            pl.BlockSpec((8, D2), lambda i: (0, 0)),
            pl.BlockSpec((BLK, D1), lambda i: (i, 0)),
        ],
        out_specs=pl.BlockSpec((BLK, D2), lambda i: (i, 0)),
        out_shape=jax.ShapeDtypeStruct((NP, D2), jnp.float32),
    )(acc0, acc1, table1, W2, b2b, dinv_b)


FBLK = 200


def _fin_body(a_ref, b_ref, t_ref, dv_ref, o_ref):
    o_ref[...] = (a_ref[...] + b_ref[...] + t_ref[...]) * dv_ref[:, :D2]


def _fin_tc(acc0, acc1, table2, dinv_b):
    return pl.pallas_call(
        _fin_body,
        grid=(N // FBLK,),
        in_specs=[
            pl.BlockSpec((FBLK, D2), lambda i: (i, 0)),
            pl.BlockSpec((FBLK, D2), lambda i: (i, 0)),
            pl.BlockSpec((FBLK, D2), lambda i: (i, 0)),
            pl.BlockSpec((FBLK, D1), lambda i: (i, 0)),
        ],
        out_specs=pl.BlockSpec((FBLK, D2), lambda i: (i, 0)),
        out_shape=jax.ShapeDtypeStruct((N, D2), jnp.float32),
    )(acc0, acc1, table2, dinv_b)


@jax.jit
def kernel(x, edge_index, W1, b1, W2, b2):
    src = edge_index[0].astype(jnp.int32)
    dst = edge_index[1].astype(jnp.int32)

    pad = EPAD - E
    src_p = jnp.concatenate([src, jnp.full((pad,), DUMP, jnp.int32)])
    dst_p = jnp.concatenate([dst, jnp.full((pad,), DUMP, jnp.int32)])
    dst3 = dst_p.reshape(NW, CPT, CH)
    srcC = src_p.reshape(TOTC, CH)
    dstC = dst_p.reshape(TOTC, CH)

    xp = jnp.zeros((NP, x.shape[1]), x.dtype).at[:N].set(x)

    cnt = _deg_sc(dst3)
    table1, dinv_b = _mm1_tc(xp, W1, b1, cnt[0], cnt[1])
    acc1 = _spmm_sc(table1, srcC, dstC, D1)
    table2 = _mm2_tc(acc1[0], acc1[1], table1, W2, b2, dinv_b)
    acc2 = _spmm_sc(table2, srcC, dstC, D2)
    return _fin_tc(acc2[0], acc2[1], table2, dinv_b)


# speedup vs baseline: 1.1709x; 1.0929x over previous
import functools

import jax
import jax.numpy as jnp
from jax import lax
from jax.experimental import pallas as pl
from jax.experimental.pallas import tpu as pltpu
from jax.experimental.pallas import tpu_sc as plsc

N = 10000
NP = 10240
E = 320000
D1 = 128
D2 = 64
BLK = 256

NC, NS, L = 2, 16, 16
NW = NC * NS
CH = 128
CPT = 80
EPAD = NW * CPT * CH
DUMP = NP - 1
RPT = NP // NS

_mesh = lambda: plsc.VectorSubcoreMesh(core_axis_name="c", subcore_axis_name="s")


def _deg_body(dst_hbm, out_hbm, dst_v, ones_v, acc_sp, *_):
    cid = lax.axis_index("c")
    sid = lax.axis_index("s")
    wid = cid * NS + sid

    ones = jnp.full((L,), 1.0, dtype=jnp.float32)

    def _fill(r, _):
        ones_v[r, pl.ds(0, L)] = ones
        return _
    lax.fori_loop(0, CH, _fill, None)

    for k in range(RPT // CH):
        pltpu.sync_copy(ones_v, acc_sp.at[pl.ds(sid * RPT + k * CH, CH)])

    pltpu.sync_copy(dst_hbm.at[wid], dst_v)
    plsc.subcore_barrier()

    def _chunk(c, _):
        pltpu.sync_copy(ones_v, acc_sp.at[dst_v.at[c]], add=True)
        return _
    lax.fori_loop(0, CPT, _chunk, None)
    plsc.subcore_barrier()

    pltpu.sync_copy(acc_sp.at[pl.ds(sid * RPT, RPT)],
                    out_hbm.at[cid, pl.ds(sid * RPT, RPT)])


def _deg_sc(dst3):
    k = pl.kernel(
        _deg_body,
        out_type=jax.ShapeDtypeStruct((NC, NP, L), jnp.float32),
        mesh=_mesh(),
        scratch_types=[
            pltpu.VMEM((CPT, CH), jnp.int32),
            pltpu.VMEM((CH, L), jnp.float32),
            pltpu.VMEM_SHARED((NP, L), jnp.float32),
        ],
    )
    return k(dst3)


def _mm1_body(x_ref, w_ref, b_ref, d0_ref, d1_ref, o_ref, dv_ref):
    i = pl.program_id(0)
    deg = d0_ref[:, 0:1] + d1_ref[:, 0:1]
    rows = i * BLK + lax.broadcasted_iota(jnp.int32, (BLK, 1), 0)
    dcol = jnp.where(rows < N, lax.rsqrt(deg - 1.0), 0.0)
    dv = jnp.broadcast_to(dcol, (BLK, D1))
    dv_ref[...] = dv
    acc = jnp.dot(x_ref[...], w_ref[...], preferred_element_type=jnp.float32)
    o_ref[...] = (acc + b_ref[0:1, :]) * dv


def _mm1_tc(xp, W1, b1, deg0, deg1):
    b1b = jnp.broadcast_to(b1[None, :], (8, D1))
    return pl.pallas_call(
        _mm1_body,
        grid=(NP // BLK,),
        in_specs=[
            pl.BlockSpec((BLK, D1), lambda i: (i, 0)),
            pl.BlockSpec((D1, D1), lambda i: (0, 0)),
            pl.BlockSpec((8, D1), lambda i: (0, 0)),
            pl.BlockSpec((BLK, L), lambda i: (i, 0)),
            pl.BlockSpec((BLK, L), lambda i: (i, 0)),
        ],
        out_specs=[
            pl.BlockSpec((BLK, D1), lambda i: (i, 0)),
            pl.BlockSpec((BLK, D1), lambda i: (i, 0)),
        ],
        out_shape=[
            jax.ShapeDtypeStruct((NP, D1), jnp.float32),
            jax.ShapeDtypeStruct((NP, D1), jnp.float32),
        ],
    )(xp, W1, b1b, deg0, deg1)


CPT0 = 96
CPT1 = 64
G = 32
TOTC = EPAD // CH


def _spmm_body(D, table_hbm, src_hbm, dst_hbm, out_hbm, src_v, dst_v,
               rows0_v, rows1_v, acc_sp, sem0, sem1, *_):
    cid = lax.axis_index("c")
    sid = lax.axis_index("s")

    zeros = jnp.zeros((L,), dtype=jnp.float32)

    def _zfill(r, _):
        for c in range(D // L):
            rows0_v[r, pl.ds(c * L, L)] = zeros
        return _
    lax.fori_loop(0, CH, _zfill, None)
    for k in range(RPT // CH):
        pltpu.sync_copy(rows0_v, acc_sp.at[pl.ds(sid * RPT + k * CH, CH)])
    plsc.subcore_barrier()

    n_groups = jnp.where(cid == 0, CPT0 // G, CPT1 // G)
    base = jnp.where(cid == 0, sid * CPT0, NS * CPT0 + sid * CPT1)

    def _group(g, _):
        pltpu.sync_copy(src_hbm.at[pl.ds(base + g * G, G)], src_v)
        pltpu.sync_copy(dst_hbm.at[pl.ds(base + g * G, G)], dst_v)
        pltpu.async_copy(table_hbm.at[src_v.at[0]], rows0_v, sem0)

        def _pair(i, _):
            a = 2 * i
            pltpu.make_async_copy(table_hbm.at[src_v.at[a]],
                                  rows0_v, sem0).wait()
            pltpu.async_copy(table_hbm.at[src_v.at[a + 1]], rows1_v, sem1)
            pltpu.sync_copy(rows0_v, acc_sp.at[dst_v.at[a]], add=True)
            pltpu.make_async_copy(table_hbm.at[src_v.at[a + 1]],
                                  rows1_v, sem1).wait()

            @pl.when(i < G // 2 - 1)
            def _():
                pltpu.async_copy(table_hbm.at[src_v.at[a + 2]], rows0_v, sem0)

            pltpu.sync_copy(rows1_v, acc_sp.at[dst_v.at[a + 1]], add=True)
            return _
        lax.fori_loop(0, G // 2, _pair, None)
        return _
    lax.fori_loop(0, n_groups, _group, None)
    plsc.subcore_barrier()

    pltpu.sync_copy(acc_sp.at[pl.ds(sid * RPT, RPT)],
                    out_hbm.at[cid, pl.ds(sid * RPT, RPT)])


def _spmm_sc(table, srcC, dstC, D):
    k = pl.kernel(
        functools.partial(_spmm_body, D),
        out_type=jax.ShapeDtypeStruct((NC, NP, D), jnp.float32),
        mesh=_mesh(),
        compiler_params=pltpu.CompilerParams(use_tc_tiling_on_sc=False)
        if D != 128 else None,
        scratch_types=[
            pltpu.VMEM((G, CH), jnp.int32),
            pltpu.VMEM((G, CH), jnp.int32),
            pltpu.VMEM((CH, D), jnp.float32),
            pltpu.VMEM((CH, D), jnp.float32),
            pltpu.VMEM_SHARED((NP, D), jnp.float32),
            pltpu.SemaphoreType.DMA,
            pltpu.SemaphoreType.DMA,
        ],
    )
    return k(table, srcC, dstC)


def _mm2_body(a_ref, b_ref, t_ref, w_ref, bias_ref, dv_ref, o_ref):
    dv = dv_ref[...]
    h = jnp.maximum((a_ref[...] + b_ref[...] + t_ref[...]) * dv, 0.0)
    acc = jnp.dot(h, w_ref[...], preferred_element_type=jnp.float32)
    o_ref[...] = (acc + bias_ref[0:1, :]) * dv[:, :D2]


def _mm2_tc(acc0, acc1, table1, W2, b2, dinv_b):
    b2b = jnp.broadcast_to(b2[None, :], (8, D2))
    return pl.pallas_call(
        _mm2_body,
        grid=(NP // BLK,),
        in_specs=[
            pl.BlockSpec((BLK, D1), lambda i: (i, 0)),
            pl.BlockSpec((BLK, D1), lambda i: (i, 0)),
            pl.BlockSpec((BLK, D1), lambda i: (i, 0)),
            pl.BlockSpec((D1, D2), lambda i: (0, 0)),
            pl.BlockSpec((8, D2), lambda i: (0, 0)),
            pl.BlockSpec((BLK, D1), lambda i: (i, 0)),
        ],
        out_specs=pl.BlockSpec((BLK, D2), lambda i: (i, 0)),
        out_shape=jax.ShapeDtypeStruct((NP, D2), jnp.float32),
    )(acc0, acc1, table1, W2, b2b, dinv_b)


FBLK = 200


def _fin_body(a_ref, b_ref, t_ref, dv_ref, o_ref):
    o_ref[...] = (a_ref[...] + b_ref[...] + t_ref[...]) * dv_ref[:, :D2]


def _fin_tc(acc0, acc1, table2, dinv_b):
    return pl.pallas_call(
        _fin_body,
        grid=(N // FBLK,),
        in_specs=[
            pl.BlockSpec((FBLK, D2), lambda i: (i, 0)),
            pl.BlockSpec((FBLK, D2), lambda i: (i, 0)),
            pl.BlockSpec((FBLK, D2), lambda i: (i, 0)),
            pl.BlockSpec((FBLK, D1), lambda i: (i, 0)),
        ],
        out_specs=pl.BlockSpec((FBLK, D2), lambda i: (i, 0)),
        out_shape=jax.ShapeDtypeStruct((N, D2), jnp.float32),
    )(acc0, acc1, table2, dinv_b)


@jax.jit
def kernel(x, edge_index, W1, b1, W2, b2):
    src = edge_index[0].astype(jnp.int32)
    dst = edge_index[1].astype(jnp.int32)

    pad = EPAD - E
    src_p = jnp.concatenate([src, jnp.full((pad,), DUMP, jnp.int32)])
    dst_p = jnp.concatenate([dst, jnp.full((pad,), DUMP, jnp.int32)])
    dst3 = dst_p.reshape(NW, CPT, CH)
    srcC = src_p.reshape(TOTC, CH)
    dstC = dst_p.reshape(TOTC, CH)

    xp = jnp.zeros((NP, x.shape[1]), x.dtype).at[:N].set(x)

    cnt = _deg_sc(dst3)
    table1, dinv_b = _mm1_tc(xp, W1, b1, cnt[0], cnt[1])
    acc1 = _spmm_sc(table1, srcC, dstC, D1)
    table2 = _mm2_tc(acc1[0], acc1[1], table1, W2, b2, dinv_b)
    acc2 = _spmm_sc(table2, srcC, dstC, D2)
    return _fin_tc(acc2[0], acc2[1], table2, dinv_b)


# speedup vs baseline: 1.2940x; 1.1052x over previous
import functools

import jax
import jax.numpy as jnp
from jax import lax
from jax.experimental import pallas as pl
from jax.experimental.pallas import tpu as pltpu
from jax.experimental.pallas import tpu_sc as plsc

N = 10000
NP = 10240
E = 320000
D1 = 128
D2 = 64
BLK = 256

NC, NS, L = 2, 16, 16
NW = NC * NS
CH = 128
CPT = 80
EPAD = NW * CPT * CH
DUMP = NP - 1
RPT = NP // NS

_mesh = lambda: plsc.VectorSubcoreMesh(core_axis_name="c", subcore_axis_name="s")


def _deg_body(dst_hbm, out_hbm, dst_v, ones_v, acc_sp, *_):
    cid = lax.axis_index("c")
    sid = lax.axis_index("s")
    wid = cid * NS + sid

    ones = jnp.full((L,), 1.0, dtype=jnp.float32)

    def _fill(r, _):
        ones_v[r, pl.ds(0, L)] = ones
        return _
    lax.fori_loop(0, CH, _fill, None)

    for k in range(RPT // CH):
        pltpu.sync_copy(ones_v, acc_sp.at[pl.ds(sid * RPT + k * CH, CH)])

    pltpu.sync_copy(dst_hbm.at[wid], dst_v)
    plsc.subcore_barrier()

    def _chunk(c, _):
        pltpu.sync_copy(ones_v, acc_sp.at[dst_v.at[c]], add=True)
        return _
    lax.fori_loop(0, CPT, _chunk, None)
    plsc.subcore_barrier()

    pltpu.sync_copy(acc_sp.at[pl.ds(sid * RPT, RPT)],
                    out_hbm.at[cid, pl.ds(sid * RPT, RPT)])


def _deg_sc(dst3):
    k = pl.kernel(
        _deg_body,
        out_type=jax.ShapeDtypeStruct((NC, NP, L), jnp.float32),
        mesh=_mesh(),
        scratch_types=[
            pltpu.VMEM((CPT, CH), jnp.int32),
            pltpu.VMEM((CH, L), jnp.float32),
            pltpu.VMEM_SHARED((NP, L), jnp.float32),
        ],
    )
    return k(dst3)


def _mm1_body(x_ref, w_ref, b_ref, d0_ref, d1_ref, o_ref, dv_ref):
    i = pl.program_id(0)
    deg = d0_ref[:, 0:1] + d1_ref[:, 0:1]
    rows = i * BLK + lax.broadcasted_iota(jnp.int32, (BLK, 1), 0)
    dcol = jnp.where(rows < N, lax.rsqrt(deg - 1.0), 0.0)
    dv = jnp.broadcast_to(dcol, (BLK, D1))
    dv_ref[...] = dv
    acc = jnp.dot(x_ref[...], w_ref[...], preferred_element_type=jnp.float32)
    o_ref[...] = (acc + b_ref[0:1, :]) * dv


def _mm1_tc(xp, W1, b1, deg0, deg1):
    b1b = jnp.broadcast_to(b1[None, :], (8, D1))
    return pl.pallas_call(
        _mm1_body,
        grid=(NP // BLK,),
        in_specs=[
            pl.BlockSpec((BLK, D1), lambda i: (i, 0)),
            pl.BlockSpec((D1, D1), lambda i: (0, 0)),
            pl.BlockSpec((8, D1), lambda i: (0, 0)),
            pl.BlockSpec((BLK, L), lambda i: (i, 0)),
            pl.BlockSpec((BLK, L), lambda i: (i, 0)),
        ],
        out_specs=[
            pl.BlockSpec((BLK, D1), lambda i: (i, 0)),
            pl.BlockSpec((BLK, D1), lambda i: (i, 0)),
        ],
        out_shape=[
            jax.ShapeDtypeStruct((NP, D1), jnp.float32),
            jax.ShapeDtypeStruct((NP, D1), jnp.float32),
        ],
    )(xp, W1, b1b, deg0, deg1)


SCH = 64
NBUF = 4
CPT0 = 192
CPT1 = 128
G = 64
TOTC = EPAD // SCH


def _spmm_body(D, table_hbm, src_hbm, dst_hbm, out_hbm, src_v, dst_v,
               rows_v, acc_sp, sems, *_):
    cid = lax.axis_index("c")
    sid = lax.axis_index("s")

    zeros = jnp.zeros((L,), dtype=jnp.float32)

    def _zfill(r, _):
        for b in range(2):
            for c in range(D // L):
                rows_v[b][r, pl.ds(c * L, L)] = zeros
        return _
    lax.fori_loop(0, SCH, _zfill, None)
    for k in range(RPT // (2 * SCH)):
        pltpu.sync_copy(rows_v[0],
                        acc_sp.at[pl.ds(sid * RPT + 2 * k * SCH, SCH)])
        pltpu.sync_copy(rows_v[1],
                        acc_sp.at[pl.ds(sid * RPT + (2 * k + 1) * SCH, SCH)])
    plsc.subcore_barrier()

    n_groups = jnp.where(cid == 0, CPT0 // G, CPT1 // G)
    base = jnp.where(cid == 0, sid * CPT0, NS * CPT0 + sid * CPT1)

    def _group(g, _):
        pltpu.sync_copy(src_hbm.at[pl.ds(base + g * G, G)], src_v)
        pltpu.sync_copy(dst_hbm.at[pl.ds(base + g * G, G)], dst_v)
        for b in range(NBUF - 1):
            pltpu.async_copy(table_hbm.at[src_v.at[b]], rows_v[b], sems[b])

        def _quad(i, _):
            for b in range(NBUF):
                c = NBUF * i + b
                pltpu.make_async_copy(table_hbm.at[src_v.at[c]],
                                      rows_v[b], sems[b]).wait()

                @pl.when(c + NBUF - 1 < G)
                def _():
                    nb = (b + NBUF - 1) % NBUF
                    pltpu.async_copy(table_hbm.at[src_v.at[c + NBUF - 1]],
                                     rows_v[nb], sems[nb])

                pltpu.sync_copy(rows_v[b], acc_sp.at[dst_v.at[c]], add=True)
            return _
        lax.fori_loop(0, G // NBUF, _quad, None)
        return _
    lax.fori_loop(0, n_groups, _group, None)
    plsc.subcore_barrier()

    pltpu.sync_copy(acc_sp.at[pl.ds(sid * RPT, RPT)],
                    out_hbm.at[cid, pl.ds(sid * RPT, RPT)])


def _spmm_sc(table, srcC, dstC, D):
    def body(table_hbm, src_hbm, dst_hbm, out_hbm, src_v, dst_v,
             r0, r1, r2, r3, acc_sp, s0, s1, s2, s3):
        _spmm_body(D, table_hbm, src_hbm, dst_hbm, out_hbm, src_v, dst_v,
                   [r0, r1, r2, r3], acc_sp, [s0, s1, s2, s3])

    k = pl.kernel(
        body,
        out_type=jax.ShapeDtypeStruct((NC, NP, D), jnp.float32),
        mesh=_mesh(),
        compiler_params=pltpu.CompilerParams(use_tc_tiling_on_sc=False)
        if D != 128 else None,
        scratch_types=[
            pltpu.VMEM((G, SCH), jnp.int32),
            pltpu.VMEM((G, SCH), jnp.int32),
            pltpu.VMEM((SCH, D), jnp.float32),
            pltpu.VMEM((SCH, D), jnp.float32),
            pltpu.VMEM((SCH, D), jnp.float32),
            pltpu.VMEM((SCH, D), jnp.float32),
            pltpu.VMEM_SHARED((NP, D), jnp.float32),
            pltpu.SemaphoreType.DMA,
            pltpu.SemaphoreType.DMA,
            pltpu.SemaphoreType.DMA,
            pltpu.SemaphoreType.DMA,
        ],
    )
    return k(table, srcC, dstC)


def _mm2_body(a_ref, b_ref, t_ref, w_ref, bias_ref, dv_ref, o_ref):
    dv = dv_ref[...]
    h = jnp.maximum((a_ref[...] + b_ref[...] + t_ref[...]) * dv, 0.0)
    acc = jnp.dot(h, w_ref[...], preferred_element_type=jnp.float32)
    o_ref[...] = (acc + bias_ref[0:1, :]) * dv[:, :D2]


def _mm2_tc(acc0, acc1, table1, W2, b2, dinv_b):
    b2b = jnp.broadcast_to(b2[None, :], (8, D2))
    return pl.pallas_call(
        _mm2_body,
        grid=(NP // BLK,),
        in_specs=[
            pl.BlockSpec((BLK, D1), lambda i: (i, 0)),
            pl.BlockSpec((BLK, D1), lambda i: (i, 0)),
            pl.BlockSpec((BLK, D1), lambda i: (i, 0)),
            pl.BlockSpec((D1, D2), lambda i: (0, 0)),
            pl.BlockSpec((8, D2), lambda i: (0, 0)),
            pl.BlockSpec((BLK, D1), lambda i: (i, 0)),
        ],
        out_specs=pl.BlockSpec((BLK, D2), lambda i: (i, 0)),
        out_shape=jax.ShapeDtypeStruct((NP, D2), jnp.float32),
    )(acc0, acc1, table1, W2, b2b, dinv_b)


FBLK = 200


def _fin_body(a_ref, b_ref, t_ref, dv_ref, o_ref):
    o_ref[...] = (a_ref[...] + b_ref[...] + t_ref[...]) * dv_ref[:, :D2]


def _fin_tc(acc0, acc1, table2, dinv_b):
    return pl.pallas_call(
        _fin_body,
        grid=(N // FBLK,),
        in_specs=[
            pl.BlockSpec((FBLK, D2), lambda i: (i, 0)),
            pl.BlockSpec((FBLK, D2), lambda i: (i, 0)),
            pl.BlockSpec((FBLK, D2), lambda i: (i, 0)),
            pl.BlockSpec((FBLK, D1), lambda i: (i, 0)),
        ],
        out_specs=pl.BlockSpec((FBLK, D2), lambda i: (i, 0)),
        out_shape=jax.ShapeDtypeStruct((N, D2), jnp.float32),
    )(acc0, acc1, table2, dinv_b)


@jax.jit
def kernel(x, edge_index, W1, b1, W2, b2):
    src = edge_index[0].astype(jnp.int32)
    dst = edge_index[1].astype(jnp.int32)

    pad = EPAD - E
    src_p = jnp.concatenate([src, jnp.full((pad,), DUMP, jnp.int32)])
    dst_p = jnp.concatenate([dst, jnp.full((pad,), DUMP, jnp.int32)])
    dst3 = dst_p.reshape(NW, CPT, CH)
    srcC = src_p.reshape(TOTC, SCH)
    dstC = dst_p.reshape(TOTC, SCH)

    xp = jnp.zeros((NP, x.shape[1]), x.dtype).at[:N].set(x)

    cnt = _deg_sc(dst3)
    table1, dinv_b = _mm1_tc(xp, W1, b1, cnt[0], cnt[1])
    acc1 = _spmm_sc(table1, srcC, dstC, D1)
    table2 = _mm2_tc(acc1[0], acc1[1], table1, W2, b2, dinv_b)
    acc2 = _spmm_sc(table2, srcC, dstC, D2)
    return _fin_tc(acc2[0], acc2[1], table2, dinv_b)
